# Initial kernel scaffold; baseline (speedup 1.0000x reference)
import jax
import jax.numpy as jnp
from jax.experimental import pallas as pl


def kernel(x, adj, W1, a1, W2, a2, W3, a3, Wd, bd):
    raise NotImplementedError("write your pallas kernel here")


# speedup vs baseline: 1.8008x; 1.8008x over previous
import functools

import jax
import jax.numpy as jnp
from jax.experimental import pallas as pl
from jax.experimental.pallas import tpu as pltpu

N = 2048
H = 6
BN = 256
NEG = -9e15


def _layer(xin, wcat_ref, asrc_ref, adst_ref, madj_ref, wh_scr, es_scr,
           edt_scr, hout_ref, fout):
    hf = H * fout
    wh = jnp.dot(xin, wcat_ref[...], preferred_element_type=jnp.float32)
    wh_scr[:, :hf] = wh
    es_scr[...] = jnp.dot(wh, asrc_ref[...], preferred_element_type=jnp.float32)
    e_dst = jnp.dot(wh, adst_ref[...], preferred_element_type=jnp.float32)
    edt_scr[...] = jnp.transpose(e_dst)

    def blk(i, _):
        rows = pl.ds(i * BN, BN)
        mask = madj_ref[rows, :].astype(jnp.int32) > 0
        es_blk = es_scr[rows, :]
        for h in range(H):
            e = es_blk[:, h:h + 1] + edt_scr[h:h + 1, :]
            e = jnp.where(e >= 0, e, 0.2 * e)
            e = jnp.where(mask, e, NEG)
            mx = jnp.max(e, axis=1, keepdims=True)
            p = jnp.exp(e - mx)
            s = jnp.sum(p, axis=1, keepdims=True)
            att = p / s
            ob = jnp.dot(att, wh_scr[:, h * fout:(h + 1) * fout],
                         preferred_element_type=jnp.float32)
            ob = jnp.where(ob > 0, ob, jnp.exp(ob) - 1.0)
            hout_ref[rows, h * fout:(h + 1) * fout] = ob
        return 0

    jax.lax.fori_loop(0, N // BN, blk, 0)


def _gnn_kernel(x_ref, madj_ref,
                wc1_ref, as1_ref, ad1_ref,
                wc2_ref, as2_ref, ad2_ref,
                wc3_ref, as3_ref, ad3_ref,
                wdt_ref, bd_ref, out_ref,
                wh_scr, es_scr, edt_scr, h1_scr, h2_scr, h3_scr):
    _layer(x_ref[...], wc1_ref, as1_ref, ad1_ref, madj_ref, wh_scr, es_scr,
           edt_scr, h1_scr, 16)
    _layer(h1_scr[...], wc2_ref, as2_ref, ad2_ref, madj_ref, wh_scr, es_scr,
           edt_scr, h2_scr, 32)
    _layer(h2_scr[...], wc3_ref, as3_ref, ad3_ref, madj_ref, wh_scr, es_scr,
           edt_scr, h3_scr, 64)
    s = jnp.sum(h3_scr[...], axis=0, keepdims=True)
    nrm = jnp.maximum(jnp.sqrt(jnp.sum(s * s)), 1e-12)
    sn = s / nrm
    out_ref[...] = jnp.sum(sn * wdt_ref[...], axis=1, keepdims=True) + bd_ref[...]


def _prep(W, a):
    Hh, fin, f = W.shape
    wcat = jnp.transpose(W, (1, 0, 2)).reshape(fin, Hh * f)
    eye = jnp.eye(Hh, dtype=W.dtype)
    asrc = (a[:, :f][:, :, None] * eye[:, None, :]).reshape(Hh * f, Hh)
    adst = (a[:, f:][:, :, None] * eye[:, None, :]).reshape(Hh * f, Hh)
    return wcat, asrc, adst


@jax.jit
def kernel(x, adj, W1, a1, W2, a2, W3, a3, Wd, bd):
    madj = (adj > 0).astype(jnp.int8)
    wc1, as1, ad1 = _prep(W1, a1)
    wc2, as2, ad2 = _prep(W2, a2)
    wc3, as3, ad3 = _prep(W3, a3)
    wdt = jnp.reshape(Wd, (1, 384))
    bd2 = jnp.reshape(bd, (1, 1))

    out = pl.pallas_call(
        _gnn_kernel,
        out_shape=jax.ShapeDtypeStruct((1, 1), jnp.float32),
        scratch_shapes=[
            pltpu.VMEM((N, 384), jnp.float32),
            pltpu.VMEM((N, H), jnp.float32),
            pltpu.VMEM((H, N), jnp.float32),
            pltpu.VMEM((N, 96), jnp.float32),
            pltpu.VMEM((N, 192), jnp.float32),
            pltpu.VMEM((N, 384), jnp.float32),
        ],
    )(x, madj, wc1, as1, ad1, wc2, as2, ad2, wc3, as3, ad3, wdt, bd2)
    return jnp.reshape(out, (1,))


# speedup vs baseline: 2.1347x; 1.1854x over previous
import functools

import jax
import jax.numpy as jnp
from jax.experimental import pallas as pl
from jax.experimental.pallas import tpu as pltpu

N = 2048
H = 6
BN = 256
NEG = -9e15


def _layer(xin, wcat_ref, asrc_ref, adst_ref, madj_ref, wh_scr, es_scr,
           edt_scr, hout_ref, fout):
    hf = H * fout
    wh = jnp.dot(xin, wcat_ref[...], preferred_element_type=jnp.float32)
    wh_scr[:, :hf] = wh
    es_scr[...] = jnp.dot(wh, asrc_ref[...], preferred_element_type=jnp.float32)
    e_dst = jnp.dot(wh, adst_ref[...], preferred_element_type=jnp.float32)
    edt_scr[...] = jnp.transpose(e_dst)

    def blk(i, _):
        rows = pl.ds(i * BN, BN)
        m32 = madj_ref[rows, :].astype(jnp.int32)
        negm = m32.astype(jnp.float32) * (-NEG) + NEG
        es_blk = es_scr[rows, :]
        for h in range(H):
            e = es_blk[:, h:h + 1] + edt_scr[h:h + 1, :]
            e = jnp.maximum(e, 0.2 * e) + negm
            mx = jnp.max(e, axis=1, keepdims=True)
            p = jnp.exp(e - mx)
            s = jnp.sum(p, axis=1, keepdims=True)
            ob = jnp.dot(p, wh_scr[:, h * fout:(h + 1) * fout],
                         preferred_element_type=jnp.float32) / s
            ob = jnp.where(ob > 0, ob, jnp.exp(ob) - 1.0)
            hout_ref[rows, h * fout:(h + 1) * fout] = ob
        return 0

    jax.lax.fori_loop(0, N // BN, blk, 0)


def _gnn_kernel(x_ref, madj_ref,
                wc1_ref, as1_ref, ad1_ref,
                wc2_ref, as2_ref, ad2_ref,
                wc3_ref, as3_ref, ad3_ref,
                wdt_ref, bd_ref, out_ref,
                wh_scr, es_scr, edt_scr, h1_scr, h2_scr, h3_scr):
    _layer(x_ref[...], wc1_ref, as1_ref, ad1_ref, madj_ref, wh_scr, es_scr,
           edt_scr, h1_scr, 16)
    _layer(h1_scr[...], wc2_ref, as2_ref, ad2_ref, madj_ref, wh_scr, es_scr,
           edt_scr, h2_scr, 32)
    _layer(h2_scr[...], wc3_ref, as3_ref, ad3_ref, madj_ref, wh_scr, es_scr,
           edt_scr, h3_scr, 64)
    s = jnp.sum(h3_scr[...], axis=0, keepdims=True)
    nrm = jnp.maximum(jnp.sqrt(jnp.sum(s * s)), 1e-12)
    sn = s / nrm
    out_ref[...] = jnp.sum(sn * wdt_ref[...], axis=1, keepdims=True) + bd_ref[...]


def _prep(W, a):
    Hh, fin, f = W.shape
    wcat = jnp.transpose(W, (1, 0, 2)).reshape(fin, Hh * f)
    eye = jnp.eye(Hh, dtype=W.dtype)
    asrc = (a[:, :f][:, :, None] * eye[:, None, :]).reshape(Hh * f, Hh)
    adst = (a[:, f:][:, :, None] * eye[:, None, :]).reshape(Hh * f, Hh)
    return wcat, asrc, adst


@jax.jit
def kernel(x, adj, W1, a1, W2, a2, W3, a3, Wd, bd):
    madj = (adj > 0).astype(jnp.int8)
    wc1, as1, ad1 = _prep(W1, a1)
    wc2, as2, ad2 = _prep(W2, a2)
    wc3, as3, ad3 = _prep(W3, a3)
    wdt = jnp.reshape(Wd, (1, 384))
    bd2 = jnp.reshape(bd, (1, 1))

    out = pl.pallas_call(
        _gnn_kernel,
        out_shape=jax.ShapeDtypeStruct((1, 1), jnp.float32),
        scratch_shapes=[
            pltpu.VMEM((N, 384), jnp.float32),
            pltpu.VMEM((N, H), jnp.float32),
            pltpu.VMEM((H, N), jnp.float32),
            pltpu.VMEM((N, 96), jnp.float32),
            pltpu.VMEM((N, 192), jnp.float32),
            pltpu.VMEM((N, 384), jnp.float32),
        ],
    )(x, madj, wc1, as1, ad1, wc2, as2, ad2, wc3, as3, ad3, wdt, bd2)
    return jnp.reshape(out, (1,))


# speedup vs baseline: 2.5658x; 1.2020x over previous
import functools

import jax
import jax.numpy as jnp
from jax.experimental import pallas as pl
from jax.experimental.pallas import tpu as pltpu

N = 2048
H = 6
BN = 256
NEG = -9e15


def _layer(xin, wcat_ref, asrc_ref, adst_ref, madj_ref, whp_scr, es_scr,
           edt_scr, hout_ref, fout):
    wh = jnp.dot(xin, wcat_ref[...], preferred_element_type=jnp.float32)
    es_scr[...] = jnp.dot(wh, asrc_ref[...], preferred_element_type=jnp.float32)
    e_dst = jnp.dot(wh, adst_ref[...], preferred_element_type=jnp.float32)
    edt_scr[...] = jnp.transpose(e_dst)
    for h in range(H):
        whp_scr[:, h * 128:h * 128 + fout] = (
            wh[:, h * fout:(h + 1) * fout].astype(jnp.bfloat16))
        whp_scr[:, h * 128 + fout:h * 128 + fout + 1] = jnp.ones(
            (N, 1), jnp.bfloat16)
    edmax = jnp.max(edt_scr[...], axis=1, keepdims=True)

    def blk(i, _):
        rows = pl.ds(i * BN, BN)
        m32 = madj_ref[rows, :].astype(jnp.int32)
        negm = m32.astype(jnp.float32) * (-NEG) + NEG
        es_blk = es_scr[rows, :]
        for h in range(H):
            t = es_blk[:, h:h + 1] + edt_scr[h:h + 1, :]
            mxc = es_blk[:, h:h + 1] + edmax[h:h + 1, :]
            mx = jnp.maximum(mxc, 0.2 * mxc)
            e = jnp.maximum(t, 0.2 * t) + (negm - mx)
            p = jnp.exp(e).astype(jnp.bfloat16)
            of = jnp.dot(p, whp_scr[:, h * 128:h * 128 + fout + 1],
                         preferred_element_type=jnp.float32)
            s = jnp.maximum(of[:, fout:fout + 1], 1e-30)
            ob = of[:, :fout] / s
            ob = jnp.where(ob > 0, ob, jnp.exp(ob) - 1.0)
            hout_ref[rows, h * fout:(h + 1) * fout] = ob
        return 0

    jax.lax.fori_loop(0, N // BN, blk, 0)


def _gnn_kernel(x_ref, madj_ref,
                wc1_ref, as1_ref, ad1_ref,
                wc2_ref, as2_ref, ad2_ref,
                wc3_ref, as3_ref, ad3_ref,
                wdt_ref, bd_ref, out_ref,
                wh_scr, es_scr, edt_scr, h1_scr, h2_scr, h3_scr):
    _layer(x_ref[...], wc1_ref, as1_ref, ad1_ref, madj_ref, wh_scr, es_scr,
           edt_scr, h1_scr, 16)
    _layer(h1_scr[...], wc2_ref, as2_ref, ad2_ref, madj_ref, wh_scr, es_scr,
           edt_scr, h2_scr, 32)
    _layer(h2_scr[...], wc3_ref, as3_ref, ad3_ref, madj_ref, wh_scr, es_scr,
           edt_scr, h3_scr, 64)
    s = jnp.sum(h3_scr[...], axis=0, keepdims=True)
    nrm = jnp.maximum(jnp.sqrt(jnp.sum(s * s)), 1e-12)
    sn = s / nrm
    out_ref[...] = jnp.sum(sn * wdt_ref[...], axis=1, keepdims=True) + bd_ref[...]


def _prep(W, a):
    Hh, fin, f = W.shape
    wcat = jnp.transpose(W, (1, 0, 2)).reshape(fin, Hh * f)
    eye = jnp.eye(Hh, dtype=W.dtype)
    asrc = (a[:, :f][:, :, None] * eye[:, None, :]).reshape(Hh * f, Hh)
    adst = (a[:, f:][:, :, None] * eye[:, None, :]).reshape(Hh * f, Hh)
    return wcat, asrc, adst


@jax.jit
def kernel(x, adj, W1, a1, W2, a2, W3, a3, Wd, bd):
    madj = (adj > 0).astype(jnp.int8)
    wc1, as1, ad1 = _prep(W1, a1)
    wc2, as2, ad2 = _prep(W2, a2)
    wc3, as3, ad3 = _prep(W3, a3)
    wdt = jnp.reshape(Wd, (1, 384))
    bd2 = jnp.reshape(bd, (1, 1))

    out = pl.pallas_call(
        _gnn_kernel,
        out_shape=jax.ShapeDtypeStruct((1, 1), jnp.float32),
        scratch_shapes=[
            pltpu.VMEM((N, H * 128), jnp.bfloat16),
            pltpu.VMEM((N, H), jnp.float32),
            pltpu.VMEM((H, N), jnp.float32),
            pltpu.VMEM((N, 96), jnp.float32),
            pltpu.VMEM((N, 192), jnp.float32),
            pltpu.VMEM((N, 384), jnp.float32),
        ],
    )(x, madj, wc1, as1, ad1, wc2, as2, ad2, wc3, as3, ad3, wdt, bd2)
    return jnp.reshape(out, (1,))


# speedup vs baseline: 2.9109x; 1.1345x over previous
import functools

import jax
import jax.numpy as jnp
from jax.experimental import pallas as pl
from jax.experimental.pallas import tpu as pltpu

N = 2048
H = 6
BN = 256
NEG = -9e15


def _layer(xin, wcat_ref, asrc_ref, adst_ref, mf_ref, whp_scr, es_scr,
           edt_scr, edt2_scr, hout_ref, fout):
    wh = jnp.dot(xin, wcat_ref[...], preferred_element_type=jnp.float32)
    es_scr[...] = jnp.dot(wh, asrc_ref[...], preferred_element_type=jnp.float32)
    e_dst = jnp.dot(wh, adst_ref[...], preferred_element_type=jnp.float32)
    edt = jnp.transpose(e_dst)
    edt_scr[...] = edt
    edt2_scr[...] = 0.2 * edt
    for h in range(H):
        whp_scr[:, h * 128:h * 128 + fout] = (
            wh[:, h * fout:(h + 1) * fout].astype(jnp.bfloat16))
        whp_scr[:, h * 128 + fout:h * 128 + fout + 1] = jnp.ones(
            (N, 1), jnp.bfloat16)
    edmax = jnp.max(edt, axis=1, keepdims=True)

    def blk(i, _):
        rows = pl.ds(i * BN, BN)
        mfb = mf_ref[rows, :]
        es_blk = es_scr[rows, :]
        for h in range(H):
            esc = es_blk[:, h:h + 1]
            mxc = esc + edmax[h:h + 1, :]
            mx = jnp.maximum(mxc, 0.2 * mxc)
            esm = esc - mx
            esm2 = 0.2 * esc - mx
            e = jnp.maximum(esm + edt_scr[h:h + 1, :],
                            esm2 + edt2_scr[h:h + 1, :])
            p = jnp.exp(e).astype(jnp.bfloat16) * mfb
            of = jnp.dot(p, whp_scr[:, h * 128:h * 128 + fout + 1],
                         preferred_element_type=jnp.float32)
            s = jnp.maximum(of[:, fout:fout + 1], 1e-30)
            ob = of[:, :fout] / s
            ob = jnp.where(ob > 0, ob, jnp.exp(ob) - 1.0)
            hout_ref[rows, h * fout:(h + 1) * fout] = ob
        return 0

    jax.lax.fori_loop(0, N // BN, blk, 0)


def _gnn_kernel(x_ref, mf_ref,
                wc1_ref, as1_ref, ad1_ref,
                wc2_ref, as2_ref, ad2_ref,
                wc3_ref, as3_ref, ad3_ref,
                wdt_ref, bd_ref, out_ref,
                wh_scr, es_scr, edt_scr, edt2_scr, h1_scr, h2_scr, h3_scr):
    _layer(x_ref[...], wc1_ref, as1_ref, ad1_ref, mf_ref, wh_scr, es_scr,
           edt_scr, edt2_scr, h1_scr, 16)
    _layer(h1_scr[...], wc2_ref, as2_ref, ad2_ref, mf_ref, wh_scr, es_scr,
           edt_scr, edt2_scr, h2_scr, 32)
    _layer(h2_scr[...], wc3_ref, as3_ref, ad3_ref, mf_ref, wh_scr, es_scr,
           edt_scr, edt2_scr, h3_scr, 64)
    s = jnp.sum(h3_scr[...], axis=0, keepdims=True)
    nrm = jnp.maximum(jnp.sqrt(jnp.sum(s * s)), 1e-12)
    sn = s / nrm
    out_ref[...] = jnp.sum(sn * wdt_ref[...], axis=1, keepdims=True) + bd_ref[...]


def _prep(W, a):
    Hh, fin, f = W.shape
    wcat = jnp.transpose(W, (1, 0, 2)).reshape(fin, Hh * f)
    eye = jnp.eye(Hh, dtype=W.dtype)
    asrc = (a[:, :f][:, :, None] * eye[:, None, :]).reshape(Hh * f, Hh)
    adst = (a[:, f:][:, :, None] * eye[:, None, :]).reshape(Hh * f, Hh)
    return wcat, asrc, adst


@jax.jit
def kernel(x, adj, W1, a1, W2, a2, W3, a3, Wd, bd):
    mf = (adj > 0).astype(jnp.bfloat16)
    wc1, as1, ad1 = _prep(W1, a1)
    wc2, as2, ad2 = _prep(W2, a2)
    wc3, as3, ad3 = _prep(W3, a3)
    wdt = jnp.reshape(Wd, (1, 384))
    bd2 = jnp.reshape(bd, (1, 1))

    out = pl.pallas_call(
        _gnn_kernel,
        out_shape=jax.ShapeDtypeStruct((1, 1), jnp.float32),
        scratch_shapes=[
            pltpu.VMEM((N, H * 128), jnp.bfloat16),
            pltpu.VMEM((N, H), jnp.float32),
            pltpu.VMEM((H, N), jnp.float32),
            pltpu.VMEM((H, N), jnp.float32),
            pltpu.VMEM((N, 96), jnp.float32),
            pltpu.VMEM((N, 192), jnp.float32),
            pltpu.VMEM((N, 384), jnp.float32),
        ],
    )(x, mf, wc1, as1, ad1, wc2, as2, ad2, wc3, as3, ad3, wdt, bd2)
    return jnp.reshape(out, (1,))
